# Initial kernel scaffold; baseline (speedup 1.0000x reference)
import jax
import jax.numpy as jnp
from jax.experimental import pallas as pl


def kernel(queries, mem_kv, mem_norm, w_sel, wq, bq, wk, bk, wv, bv, wo, bo):
    raise NotImplementedError("write your pallas kernel here")


# speedup vs baseline: 2.1065x; 2.1065x over previous
import functools

import jax
import jax.numpy as jnp
from jax.experimental import pallas as pl
from jax.experimental.pallas import tpu as pltpu

N, D, H, E, K = 2048, 768, 64, 64, 2
EPS = 1e-6
NEG = -1e30


def _body(x_ref, wsel_ref, wq_ref, bq_ref, wk_ref, bk_ref, wv_ref, bv_ref,
          wo_ref, bo_ref, mkv_ref, mnorm_ref,
          out_ref, nkv_ref, nnorm_ref,
          i1_ref, i2_ref, g1_ref, g2_ref):
    h = pl.program_id(0)

    @pl.when(h == 0)
    def _routing():
        x = x_ref[...]
        logits = jnp.dot(x, wsel_ref[...], preferred_element_type=jnp.float32)
        ii = jax.lax.broadcasted_iota(jnp.int32, (N, H), 1)
        m1 = jnp.max(logits, axis=1, keepdims=True)
        i1 = jnp.min(jnp.where(logits == m1, ii, H), axis=1, keepdims=True)
        l2 = jnp.where(ii == i1, NEG, logits)
        m2 = jnp.max(l2, axis=1, keepdims=True)
        i2 = jnp.min(jnp.where(l2 == m2, ii, H), axis=1, keepdims=True)
        z = jnp.sum(jnp.exp(logits - m1), axis=1, keepdims=True)
        p1 = 1.0 / z
        p2 = jnp.exp(m2 - m1) / z
        s = p1 + p2 + EPS
        i1_ref[...] = i1
        i2_ref[...] = i2
        g1_ref[...] = p1 / s
        g2_ref[...] = p2 / s
        out_ref[...] = jnp.zeros_like(out_ref)

    x = x_ref[...]
    selh = (jnp.where(i1_ref[...] == h, g1_ref[...], 0.0)
            + jnp.where(i2_ref[...] == h, g2_ref[...], 0.0))

    q = jnp.dot(x, wq_ref[0], preferred_element_type=jnp.float32) + bq_ref[0]
    k = jnp.dot(x, wk_ref[0], preferred_element_type=jnp.float32) + bk_ref[0]
    v = jnp.dot(x, wv_ref[0], preferred_element_type=jnp.float32) + bv_ref[0]

    kg = k * selh
    kv_upd = jax.lax.dot_general(kg, v, (((0,), (0,)), ((), ())),
                                 preferred_element_type=jnp.float32)
    nkv = mkv_ref[0] + kv_upd
    nnorm = mnorm_ref[0] + jnp.sum(kg, axis=0, keepdims=True)
    nkv_ref[0] = nkv
    nnorm_ref[0] = nnorm

    num = jnp.dot(q, nkv, preferred_element_type=jnp.float32)
    den = jnp.sum(q * nnorm, axis=1, keepdims=True) + EPS
    attn_g = jnp.where(selh != 0.0, num / den * selh, 0.0)
    outc = jnp.dot(attn_g, wo_ref[0], preferred_element_type=jnp.float32)
    outc = outc + selh * bo_ref[0]
    out_ref[...] += outc


@jax.jit
def kernel(queries, mem_kv, mem_norm, w_sel, wq, bq, wk, bk, wv, bv, wo, bo):
    grid = (H,)
    out, nkv, nnorm = pl.pallas_call(
        _body,
        grid=grid,
        in_specs=[
            pl.BlockSpec((N, D), lambda h: (0, 0)),
            pl.BlockSpec((D, H), lambda h: (0, 0)),
            pl.BlockSpec((1, D, E), lambda h: (h, 0, 0)),
            pl.BlockSpec((1, 1, E), lambda h: (h, 0, 0)),
            pl.BlockSpec((1, D, E), lambda h: (h, 0, 0)),
            pl.BlockSpec((1, 1, E), lambda h: (h, 0, 0)),
            pl.BlockSpec((1, D, E), lambda h: (h, 0, 0)),
            pl.BlockSpec((1, 1, E), lambda h: (h, 0, 0)),
            pl.BlockSpec((1, E, D), lambda h: (h, 0, 0)),
            pl.BlockSpec((1, 1, D), lambda h: (h, 0, 0)),
            pl.BlockSpec((1, E, E), lambda h: (h, 0, 0)),
            pl.BlockSpec((1, 1, E), lambda h: (h, 0, 0)),
        ],
        out_specs=[
            pl.BlockSpec((N, D), lambda h: (0, 0)),
            pl.BlockSpec((1, E, E), lambda h: (h, 0, 0)),
            pl.BlockSpec((1, 1, E), lambda h: (h, 0, 0)),
        ],
        out_shape=[
            jax.ShapeDtypeStruct((N, D), jnp.float32),
            jax.ShapeDtypeStruct((H, E, E), jnp.float32),
            jax.ShapeDtypeStruct((H, 1, E), jnp.float32),
        ],
        scratch_shapes=[
            pltpu.VMEM((N, 1), jnp.int32),
            pltpu.VMEM((N, 1), jnp.int32),
            pltpu.VMEM((N, 1), jnp.float32),
            pltpu.VMEM((N, 1), jnp.float32),
        ],
    )(queries, w_sel, wq, bq.reshape(H, 1, E), wk, bk.reshape(H, 1, E),
      wv, bv.reshape(H, 1, E), wo, bo.reshape(H, 1, D), mem_kv,
      mem_norm.reshape(H, 1, E))
    return out, nkv, nnorm.reshape(H, E)


# speedup vs baseline: 3.2786x; 1.5564x over previous
import functools

import jax
import jax.numpy as jnp
from jax import lax
from jax.experimental import pallas as pl
from jax.experimental.pallas import tpu as pltpu
from jax.experimental.pallas import tpu_sc as plsc

N, D, H, E, K = 2048, 768, 64, 64, 2
BS = 128
NBLK = H + N * K // BS
PT = NBLK * BS
CH = N // BS
EPS = 1e-6
NEG = -1e30

NC, NS = 2, 16
NW = NC * NS
TPW = N // NW


def _routing_body(x_ref, wsel_ref, pos1_ref, pos2_ref, g1_ref, g2_ref,
                  hb_ref, vd_ref, fr_ref, c_scr, s_scr):
    x = x_ref[...]
    logits = jnp.dot(x, wsel_ref[...], preferred_element_type=jnp.float32)
    ii = lax.broadcasted_iota(jnp.int32, (N, H), 1)
    m1 = jnp.max(logits, axis=1, keepdims=True)
    i1 = jnp.min(jnp.where(logits == m1, ii, H), axis=1, keepdims=True)
    l2 = jnp.where(ii == i1, NEG, logits)
    m2 = jnp.max(l2, axis=1, keepdims=True)
    i2 = jnp.min(jnp.where(l2 == m2, ii, H), axis=1, keepdims=True)
    z = jnp.sum(jnp.exp(logits - m1), axis=1, keepdims=True)
    p1 = 1.0 / z
    p2 = jnp.exp(m2 - m1) / z
    s = p1 + p2 + EPS
    ones16 = jnp.ones((1, 128), jnp.float32)
    g1_ref[...] = (p1 / s) * ones16
    g2_ref[...] = (p2 / s) * ones16

    oh1 = (ii == i1).astype(jnp.float32)
    oh2 = (ii == i2).astype(jnp.float32)
    c = oh1 + oh2
    c_scr[...] = c
    cnt = jnp.sum(c, axis=0, keepdims=True)
    nb = jnp.maximum(jnp.ceil(cnt * (1.0 / BS)), 1.0)
    hh0 = lax.broadcasted_iota(jnp.int32, (H, H), 0)
    hh1 = lax.broadcasted_iota(jnp.int32, (H, H), 1)
    upper = (hh0 < hh1).astype(jnp.float32)
    bsf = jnp.dot(nb, upper, preferred_element_type=jnp.float32)

    rr0 = lax.broadcasted_iota(jnp.int32, (BS, BS), 0)
    rr1 = lax.broadcasted_iota(jnp.int32, (BS, BS), 1)
    tril = (rr1 < rr0).astype(jnp.float32)

    def chunk(cix, base):
        blk = c_scr[pl.ds(cix * BS, BS), :]
        s_scr[pl.ds(cix * BS, BS), :] = (
            jnp.dot(tril, blk, preferred_element_type=jnp.float32) + base)
        return base + jnp.sum(blk, axis=0, keepdims=True)

    lax.fori_loop(0, CH, chunk, jnp.zeros((1, H), jnp.float32))
    srank = s_scr[...]
    rank1 = jnp.sum(srank * oh1, axis=1, keepdims=True)
    rank2 = jnp.sum(srank * oh2, axis=1, keepdims=True)
    bs1 = jnp.sum(bsf * oh1, axis=1, keepdims=True)
    bs2 = jnp.sum(bsf * oh2, axis=1, keepdims=True)
    pos1_ref[...] = (bs1 * BS + rank1).astype(jnp.int32)
    pos2_ref[...] = (bs2 * BS + rank2).astype(jnp.int32)

    bb = lax.broadcasted_iota(jnp.int32, (NBLK, H), 0).astype(jnp.float32)
    hb = jnp.sum((bsf <= bb).astype(jnp.float32), axis=1, keepdims=True) - 1.0
    bh = lax.broadcasted_iota(jnp.int32, (NBLK, H), 1).astype(jnp.float32)
    ohb = (bh == hb).astype(jnp.float32)
    bs_at = jnp.sum(bsf * ohb, axis=1, keepdims=True)
    cnt_at = jnp.sum(cnt * ohb, axis=1, keepdims=True)
    bvals = bb[:, :1]
    base_rows = (bvals - bs_at) * BS
    vd = jnp.clip(cnt_at - base_rows, 0.0, float(BS))
    hb_ref[...] = hb.astype(jnp.int32)
    vd_ref[...] = vd.astype(jnp.int32)
    fr_ref[...] = (bvals == bs_at).astype(jnp.int32)


def _routing(x, w_sel):
    outs = pl.pallas_call(
        _routing_body,
        grid=(1,),
        in_specs=[pl.BlockSpec((N, D), lambda i: (0, 0)),
                  pl.BlockSpec((D, H), lambda i: (0, 0))],
        out_specs=[pl.BlockSpec((N, 1), lambda i: (0, 0))] * 2
        + [pl.BlockSpec((N, 128), lambda i: (0, 0))] * 2
        + [pl.BlockSpec((NBLK, 1), lambda i: (0, 0))] * 3,
        out_shape=[jax.ShapeDtypeStruct((N, 1), jnp.int32),
                   jax.ShapeDtypeStruct((N, 1), jnp.int32),
                   jax.ShapeDtypeStruct((N, 128), jnp.float32),
                   jax.ShapeDtypeStruct((N, 128), jnp.float32),
                   jax.ShapeDtypeStruct((NBLK, 1), jnp.int32),
                   jax.ShapeDtypeStruct((NBLK, 1), jnp.int32),
                   jax.ShapeDtypeStruct((NBLK, 1), jnp.int32)],
        scratch_shapes=[pltpu.VMEM((N, H), jnp.float32),
                        pltpu.VMEM((N, H), jnp.float32)],
    )(x, w_sel)
    return outs


def _scatter_body(x_hbm, p1_hbm, p2_hbm, g1_hbm, g2_hbm, xg_hbm, gg_hbm,
                  xtok, idxa, idxb, gba, gbb, sem):
    wid = lax.axis_index("s") * NC + lax.axis_index("c")
    base = wid * TPW
    pltpu.sync_copy(p1_hbm.at[pl.ds(base, TPW)], idxa)
    pltpu.sync_copy(p2_hbm.at[pl.ds(base, TPW)], idxb)
    pltpu.sync_copy(x_hbm.at[pl.ds(base, TPW)], xtok)
    pltpu.sync_copy(g1_hbm.at[pl.ds(base, TPW)], gba)
    pltpu.sync_copy(g2_hbm.at[pl.ds(base, TPW)], gbb)
    pltpu.async_copy(xtok, xg_hbm.at[idxa], sem).wait()
    pltpu.async_copy(xtok, xg_hbm.at[idxb], sem).wait()
    pltpu.async_copy(gba, gg_hbm.at[idxa], sem).wait()
    pltpu.async_copy(gbb, gg_hbm.at[idxb], sem).wait()


def _scatter(x, p1, p2, g1w, g2w):
    mesh = plsc.VectorSubcoreMesh(core_axis_name="c", subcore_axis_name="s")
    f = pl.kernel(
        _scatter_body,
        out_type=[jax.ShapeDtypeStruct((PT, D), jnp.float32),
                  jax.ShapeDtypeStruct((PT, 128), jnp.float32)],
        mesh=mesh,
        scratch_types=[pltpu.VMEM((TPW, D), jnp.float32),
                       pltpu.VMEM((TPW,), jnp.int32),
                       pltpu.VMEM((TPW,), jnp.int32),
                       pltpu.VMEM((TPW, 128), jnp.float32),
                       pltpu.VMEM((TPW, 128), jnp.float32),
                       pltpu.SemaphoreType.DMA],
    )
    return f(x, p1, p2, g1w, g2w)


def _pass_a_body(hb_ref, vd_ref, fr_ref, xg_ref, gg_ref, wq_ref, bq_ref,
                 wk_ref, bk_ref, wv_ref, bv_ref, mkv_ref, mnm_ref,
                 qg_ref, nkv_ref, nnm_ref):
    b = pl.program_id(0)
    vd = vd_ref[b]
    rows = lax.broadcasted_iota(jnp.int32, (BS, 1), 0)
    rmask = rows < vd
    xb = jnp.where(rmask, xg_ref[...], 0.0)
    g = jnp.where(rmask, gg_ref[...][:, 0:1], 0.0)
    q = jnp.dot(xb, wq_ref[0], preferred_element_type=jnp.float32) + bq_ref[0]
    k = jnp.dot(xb, wk_ref[0], preferred_element_type=jnp.float32) + bk_ref[0]
    v = jnp.dot(xb, wv_ref[0], preferred_element_type=jnp.float32) + bv_ref[0]
    qg_ref[...] = q
    kg = k * g
    kvc = lax.dot_general(kg, v, (((0,), (0,)), ((), ())),
                          preferred_element_type=jnp.float32)
    nmc = jnp.sum(kg, axis=0, keepdims=True)

    @pl.when(fr_ref[b] == 1)
    def _init():
        nkv_ref[0] = mkv_ref[0] + kvc
        nnm_ref[0] = mnm_ref[0] + nmc

    @pl.when(fr_ref[b] == 0)
    def _acc():
        nkv_ref[0] += kvc
        nnm_ref[0] += nmc


def _pass_a(hb, vd, fr, xg, gg2, wq, bq3, wk, bk3, wv, bv3, mkv, mnm3):
    return pl.pallas_call(
        _pass_a_body,
        grid_spec=pltpu.PrefetchScalarGridSpec(
            num_scalar_prefetch=3,
            grid=(NBLK,),
            in_specs=[
                pl.BlockSpec((BS, D), lambda b, hb, vd, fr: (b, 0)),
                pl.BlockSpec((BS, 128), lambda b, hb, vd, fr: (b, 0)),
                pl.BlockSpec((1, D, E), lambda b, hb, vd, fr: (hb[b], 0, 0)),
                pl.BlockSpec((1, 1, E), lambda b, hb, vd, fr: (hb[b], 0, 0)),
                pl.BlockSpec((1, D, E), lambda b, hb, vd, fr: (hb[b], 0, 0)),
                pl.BlockSpec((1, 1, E), lambda b, hb, vd, fr: (hb[b], 0, 0)),
                pl.BlockSpec((1, D, E), lambda b, hb, vd, fr: (hb[b], 0, 0)),
                pl.BlockSpec((1, 1, E), lambda b, hb, vd, fr: (hb[b], 0, 0)),
                pl.BlockSpec((1, E, E), lambda b, hb, vd, fr: (hb[b], 0, 0)),
                pl.BlockSpec((1, 1, E), lambda b, hb, vd, fr: (hb[b], 0, 0)),
            ],
            out_specs=[
                pl.BlockSpec((BS, E), lambda b, hb, vd, fr: (b, 0)),
                pl.BlockSpec((1, E, E), lambda b, hb, vd, fr: (hb[b], 0, 0)),
                pl.BlockSpec((1, 1, E), lambda b, hb, vd, fr: (hb[b], 0, 0)),
            ],
        ),
        out_shape=[jax.ShapeDtypeStruct((PT, E), jnp.float32),
                   jax.ShapeDtypeStruct((H, E, E), jnp.float32),
                   jax.ShapeDtypeStruct((H, 1, E), jnp.float32)],
    )(hb, vd, fr, xg, gg2, wq, bq3, wk, bk3, wv, bv3, mkv, mnm3)


def _pass_b_body(hb_ref, vd_ref, qg_ref, gg_ref, nkv_ref, nnm_ref,
                 wo_ref, bo_ref, yg_ref):
    b = pl.program_id(0)
    vd = vd_ref[b]
    rows = lax.broadcasted_iota(jnp.int32, (BS, 1), 0)
    rmask = rows < vd
    g = jnp.where(rmask, gg_ref[...][:, 0:1], 0.0)
    q = qg_ref[...]
    num = jnp.dot(q, nkv_ref[0], preferred_element_type=jnp.float32)
    den = jnp.sum(q * nnm_ref[0], axis=1, keepdims=True) + EPS
    attn_g = jnp.where(g != 0.0, num / den * g, 0.0)
    yg = jnp.dot(attn_g, wo_ref[0], preferred_element_type=jnp.float32)
    yg_ref[...] = yg + g * bo_ref[0]


def _pass_b(hb, vd, qg, gg2, nkv, nnm3, wo, bo3):
    return pl.pallas_call(
        _pass_b_body,
        grid_spec=pltpu.PrefetchScalarGridSpec(
            num_scalar_prefetch=2,
            grid=(NBLK,),
            in_specs=[
                pl.BlockSpec((BS, E), lambda b, hb, vd: (b, 0)),
                pl.BlockSpec((BS, 128), lambda b, hb, vd: (b, 0)),
                pl.BlockSpec((1, E, E), lambda b, hb, vd: (hb[b], 0, 0)),
                pl.BlockSpec((1, 1, E), lambda b, hb, vd: (hb[b], 0, 0)),
                pl.BlockSpec((1, E, D), lambda b, hb, vd: (hb[b], 0, 0)),
                pl.BlockSpec((1, 1, D), lambda b, hb, vd: (hb[b], 0, 0)),
            ],
            out_specs=[pl.BlockSpec((BS, D), lambda b, hb, vd: (b, 0))],
        ),
        out_shape=[jax.ShapeDtypeStruct((PT, D), jnp.float32)],
    )(hb, vd, qg, gg2, nkv, nnm3, wo, bo3)[0]


def _combine_body(yg_hbm, p1_hbm, p2_hbm, out_hbm, idxa, idxb, buf0, buf1, sem):
    wid = lax.axis_index("s") * NC + lax.axis_index("c")
    base = wid * TPW
    pltpu.sync_copy(p1_hbm.at[pl.ds(base, TPW)], idxa)
    pltpu.sync_copy(p2_hbm.at[pl.ds(base, TPW)], idxb)
    pltpu.async_copy(yg_hbm.at[idxa], buf0, sem).wait()
    pltpu.async_copy(yg_hbm.at[idxb], buf1, sem).wait()

    def row(r, _):
        def col(cix, _2):
            sl = pl.ds(cix * 16, 16)
            buf0[r, sl] = buf0[r, sl] + buf1[r, sl]
            return 0
        lax.fori_loop(0, D // 16, col, 0)
        return 0

    lax.fori_loop(0, TPW, row, 0)
    pltpu.sync_copy(buf0, out_hbm.at[pl.ds(base, TPW)])


def _combine(yg, p1, p2):
    mesh = plsc.VectorSubcoreMesh(core_axis_name="c", subcore_axis_name="s")
    f = pl.kernel(
        _combine_body,
        out_type=[jax.ShapeDtypeStruct((N, D), jnp.float32)],
        mesh=mesh,
        scratch_types=[pltpu.VMEM((TPW,), jnp.int32),
                       pltpu.VMEM((TPW,), jnp.int32),
                       pltpu.VMEM((TPW, D), jnp.float32),
                       pltpu.VMEM((TPW, D), jnp.float32),
                       pltpu.SemaphoreType.DMA],
    )
    return f(yg, p1, p2)[0]


@jax.jit
def kernel(queries, mem_kv, mem_norm, w_sel, wq, bq, wk, bk, wv, bv, wo, bo):
    p1, p2, g1, g2, hb, vd, fr = _routing(queries, w_sel)
    p1f, p2f = p1.reshape(N), p2.reshape(N)
    xg, gg = _scatter(queries, p1f, p2f, g1, g2)
    hbf, vdf, frf = hb.reshape(NBLK), vd.reshape(NBLK), fr.reshape(NBLK)
    qg, nkv, nnm = _pass_a(hbf, vdf, frf, xg, gg,
                           wq, bq.reshape(H, 1, E), wk, bk.reshape(H, 1, E),
                           wv, bv.reshape(H, 1, E), mem_kv,
                           mem_norm.reshape(H, 1, E))
    yg = _pass_b(hbf, vdf, qg, gg, nkv, nnm, wo,
                 bo.reshape(H, 1, D))
    out = _combine(yg, p1f, p2f)
    return out, nkv, nnm.reshape(H, E)
